# Initial kernel scaffold; baseline (speedup 1.0000x reference)
import jax
import jax.numpy as jnp
from jax.experimental import pallas as pl


def kernel(inputs, A, emb, Wc, bc, Win, bin_, Wout, bout, Wih, bih, Whh, bhh):
    raise NotImplementedError("write your pallas kernel here")


# speedup vs baseline: 1.9701x; 1.9701x over previous
import functools

import jax
import jax.numpy as jnp
from jax import lax
from jax.experimental import pallas as pl
from jax.experimental.pallas import tpu as pltpu
from jax.experimental.pallas import tpu_sc as plsc

D = 64
K = 4
C = 16
L = 50
ITER = 2

_NC = 2
_NS = 16
_NW = _NC * _NS
_CHUNK = 80


def _sc_gather(emb, ids3):
    n_chunk = ids3.shape[1]
    rpw = n_chunk * _CHUNK
    total = _NW * rpw
    mesh = plsc.VectorSubcoreMesh(core_axis_name="c", subcore_axis_name="s")

    @functools.partial(
        pl.kernel,
        mesh=mesh,
        out_type=jax.ShapeDtypeStruct((total, D), jnp.float32),
        scratch_types=[
            pltpu.VMEM((n_chunk, _CHUNK), jnp.int32),
            pltpu.VMEM((rpw, D), jnp.float32),
            pltpu.SemaphoreType.DMA,
        ],
        compiler_params=pltpu.CompilerParams(use_tc_tiling_on_sc=False),
    )
    def gather_kernel(emb_hbm, idx_hbm, out_hbm, idx_v, rows_v, sem):
        wid = lax.axis_index("s") * _NC + lax.axis_index("c")
        pltpu.sync_copy(idx_hbm.at[wid], idx_v)
        copies = []
        for c in range(n_chunk):
            copies.append(
                pltpu.async_copy(
                    emb_hbm.at[idx_v.at[c]],
                    rows_v.at[pl.ds(c * _CHUNK, _CHUNK)],
                    sem,
                )
            )
        for cp in copies:
            cp.wait()
        pltpu.sync_copy(rows_v, out_hbm.at[pl.ds(wid * rpw, rpw)])

    return gather_kernel(emb, ids3)


_NB = 16


def _tc_body(a_ref, hid_ref,
             wc_ref, bc_ref, g_ref,
             win_ref, bin_ref, wout_ref, bout_ref,
             war_ref, wor_ref, whr_ref, br_ref,
             waz_ref, woz_ref, whz_ref, bz_ref,
             wan_ref, won_ref, bn_ref, whn_ref, bhn_ref,
             out_ref, cor_ref,
             h_sc, pin_sc, pout_sc, ai_sc, ao_sc):
    f32 = jnp.float32

    def mm(x, w_ref):
        return jnp.dot(x, w_ref[...], preferred_element_type=f32)

    hid = hid_ref[...]
    hk = jnp.tanh(mm(hid, wc_ref) + bc_ref[...])
    ssq = jnp.dot(hk * hk, g_ref[...], preferred_element_type=f32)
    hk = hk / (jnp.sqrt(ssq) + 1e-8)
    for k in range(K):
        cor_ref[k, :, :] = hk[:, k * C:(k + 1) * C]
    h_sc[...] = hk

    for _ in range(ITER):
        h = h_sc[...]
        pin_sc[...] = mm(h, win_ref) + bin_ref[...]
        pout_sc[...] = mm(h, wout_ref) + bout_ref[...]
        for s in range(_NB):
            a_s = a_ref[s]
            rows = pl.ds(s * L, L)
            ai_sc[rows, :] = jnp.dot(a_s[:, :L], pin_sc[rows, :],
                                     preferred_element_type=f32)
            ao_sc[rows, :] = jnp.dot(a_s[:, L:], pout_sc[rows, :],
                                     preferred_element_type=f32)
        ai = ai_sc[...]
        ao = ao_sc[...]
        r = jax.nn.sigmoid(mm(ai, war_ref) + mm(ao, wor_ref)
                           + mm(h, whr_ref) + br_ref[...])
        z = jax.nn.sigmoid(mm(ai, waz_ref) + mm(ao, woz_ref)
                           + mm(h, whz_ref) + bz_ref[...])
        hn = mm(h, whn_ref) + bhn_ref[...]
        n = jnp.tanh(mm(ai, wan_ref) + mm(ao, won_ref) + bn_ref[...]
                     + r * hn)
        h_sc[...] = (1.0 - z) * n + z * h
    out_ref[...] = h_sc[...]


def _block_diag(w):
    eye = jnp.eye(K, dtype=w.dtype)
    t = w[:, :, None, :] * eye[:, None, :, None]
    return t.reshape(K * w.shape[1], K * w.shape[2])


def _pack_weights(Wc, bc, Win, bin_, Wout, bout, Wih, bih, Whh, bhh):
    wc_all = Wc.transpose(1, 0, 2).reshape(D, K * C)
    bc2 = bc.reshape(1, K * C)
    gmask = jnp.kron(jnp.eye(K, dtype=jnp.float32),
                     jnp.ones((C, C), jnp.float32))
    win_bd = _block_diag(Win)
    bin2 = bin_.reshape(1, K * C)
    wout_bd = _block_diag(Wout)
    bout2 = bout.reshape(1, K * C)

    def gate(g):
        wa = _block_diag(Wih[:, :C, g * C:(g + 1) * C])
        wo = _block_diag(Wih[:, C:, g * C:(g + 1) * C])
        wh = _block_diag(Whh[:, :, g * C:(g + 1) * C])
        bi = bih[:, g * C:(g + 1) * C].reshape(1, K * C)
        bh = bhh[:, g * C:(g + 1) * C].reshape(1, K * C)
        return wa, wo, wh, bi, bh

    war, wor, whr, bir, bhr = gate(0)
    waz, woz, whz, biz, bhz = gate(1)
    wan, won, whn, bin_n, bhn = gate(2)
    return (wc_all, bc2, gmask, win_bd, bin2, wout_bd, bout2,
            war, wor, whr, bir + bhr,
            waz, woz, whz, biz + bhz,
            wan, won, bin_n, whn, bhn)


def _dense(A, gathered, packed):
    B = A.shape[0]
    R = _NB * L
    grid = B // _NB
    rows_total = B * L
    f32 = jnp.float32

    def wspec(shape):
        nd = len(shape)
        return pl.BlockSpec(shape, lambda i, _n=nd: (0,) * _n)

    in_specs = [
        pl.BlockSpec((_NB, L, 2 * L), lambda i: (i, 0, 0)),
        pl.BlockSpec((R, D), lambda i: (i, 0)),
    ] + [wspec(p.shape) for p in packed]

    out_specs = [
        pl.BlockSpec((R, D), lambda i: (i, 0)),
        pl.BlockSpec((K, R, C), lambda i: (0, i, 0)),
    ]

    out, cor = pl.pallas_call(
        _tc_body,
        grid=(grid,),
        in_specs=in_specs,
        out_specs=out_specs,
        out_shape=[
            jax.ShapeDtypeStruct((rows_total, D), f32),
            jax.ShapeDtypeStruct((K, rows_total, C), f32),
        ],
        scratch_shapes=[pltpu.VMEM((R, D), f32) for _ in range(5)],
    )(A, gathered, *packed)
    return out, cor


def kernel(inputs, A, emb, Wc, bc, Win, bin_, Wout, bout, Wih, bih, Whh, bhh):
    B, Ls = inputs.shape
    rows = B * Ls
    rpw = rows // _NW
    n_chunk = rpw // _CHUNK
    ids3 = inputs.astype(jnp.int32).reshape(_NW, n_chunk, _CHUNK)
    gathered = _sc_gather(emb, ids3)
    packed = _pack_weights(Wc, bc, Win, bin_, Wout, bout, Wih, bih, Whh, bhh)
    out, cor = _dense(A, gathered, packed)
    hidden_out = out.reshape(B, Ls, D)
    cor_hidden = cor.reshape(K, B, Ls, C)
    return hidden_out, cor_hidden


# speedup vs baseline: 2.5210x; 1.2796x over previous
import functools

import jax
import jax.numpy as jnp
from jax import lax
from jax.experimental import pallas as pl
from jax.experimental.pallas import tpu as pltpu
from jax.experimental.pallas import tpu_sc as plsc

D = 64
K = 4
C = 16
L = 50
ITER = 2

_NC = 2
_NS = 16
_NW = _NC * _NS
_CHUNK = 80


def _sc_gather(emb, ids3):
    n_chunk = ids3.shape[1]
    rpw = n_chunk * _CHUNK
    total = _NW * rpw
    mesh = plsc.VectorSubcoreMesh(core_axis_name="c", subcore_axis_name="s")

    @functools.partial(
        pl.kernel,
        mesh=mesh,
        out_type=jax.ShapeDtypeStruct((total, D), jnp.float32),
        scratch_types=[
            pltpu.VMEM((n_chunk, _CHUNK), jnp.int32),
            pltpu.VMEM((rpw, D), jnp.float32),
            pltpu.SemaphoreType.DMA,
        ],
        compiler_params=pltpu.CompilerParams(use_tc_tiling_on_sc=False),
    )
    def gather_kernel(emb_hbm, idx_hbm, out_hbm, idx_v, rows_v, sem):
        wid = lax.axis_index("s") * _NC + lax.axis_index("c")
        pltpu.sync_copy(idx_hbm.at[wid], idx_v)
        copies = []
        for c in range(n_chunk):
            copies.append(
                pltpu.async_copy(
                    emb_hbm.at[idx_v.at[c]],
                    rows_v.at[pl.ds(c * _CHUNK, _CHUNK)],
                    sem,
                )
            )
        for cp in copies:
            cp.wait()
        pltpu.sync_copy(rows_v, out_hbm.at[pl.ds(wid * rpw, rpw)])

    return gather_kernel(emb, ids3)


_NB = 16


def _tc_body(a_ref, hid_ref,
             wc_ref, bc_ref, g_ref,
             wio_ref, bio_ref, wih_ref, bih_ref, whh_ref, bhh_ref,
             out_ref, cor_ref,
             h_sc, p_sc, x_sc):
    f32 = jnp.float32
    R = _NB * L

    def mm(x, w_ref):
        return jnp.dot(x, w_ref[...], preferred_element_type=f32)

    hid = hid_ref[...]
    hk = jnp.tanh(mm(hid, wc_ref) + bc_ref[...])
    ssq = jnp.dot(hk * hk, g_ref[...], preferred_element_type=f32)
    hk = hk / (jnp.sqrt(ssq) + 1e-8)
    for k in range(K):
        cor_ref[k] = jnp.reshape(hk[:, k * C:(k + 1) * C], (_NB, L, C))
    h_sc[...] = hk

    col = lax.broadcasted_iota(jnp.int32, (L, 2 * D), 1)
    mlo = (col < D).astype(f32)
    mhi = 1.0 - mlo
    for _ in range(ITER):
        h = h_sc[...]
        p_sc[...] = mm(h, wio_ref) + bio_ref[...]
        for s in range(_NB):
            p_s = p_sc[pl.ds(s * L, L), :]
            pp = jnp.concatenate([p_s * mlo, p_s * mhi], axis=0)
            x_sc[pl.ds(s * L, L), :] = jnp.dot(a_ref[s], pp,
                                               preferred_element_type=f32)
        gi = mm(x_sc[...], wih_ref) + bih_ref[...]
        gh = mm(h, whh_ref) + bhh_ref[...]
        r = jax.nn.sigmoid(gi[:, :D] + gh[:, :D])
        z = jax.nn.sigmoid(gi[:, D:2 * D] + gh[:, D:2 * D])
        n = jnp.tanh(gi[:, 2 * D:] + r * gh[:, 2 * D:])
        h_sc[...] = (1.0 - z) * n + z * h
    out_ref[...] = jnp.reshape(h_sc[...], (_NB, L, D))


def _block_diag(w):
    eye = jnp.eye(K, dtype=w.dtype)
    t = w[:, :, None, :] * eye[:, None, :, None]
    return t.reshape(K * w.shape[1], K * w.shape[2])


def _pack_weights(Wc, bc, Win, bin_, Wout, bout, Wih, bih, Whh, bhh):
    wc_all = Wc.transpose(1, 0, 2).reshape(D, K * C)
    bc2 = bc.reshape(1, K * C)
    gmask = jnp.kron(jnp.eye(K, dtype=jnp.float32),
                     jnp.ones((C, C), jnp.float32))
    wio = jnp.concatenate([_block_diag(Win), _block_diag(Wout)], axis=1)
    bio = jnp.concatenate([bin_.reshape(1, K * C), bout.reshape(1, K * C)],
                          axis=1)

    def gates(w):
        return jnp.concatenate(
            [_block_diag(w[:, :, g * C:(g + 1) * C]) for g in range(3)],
            axis=1)

    wih_p = jnp.concatenate([gates(Wih[:, :C, :]), gates(Wih[:, C:, :])],
                            axis=0)
    bih_p = jnp.concatenate(
        [bih[:, g * C:(g + 1) * C].reshape(1, K * C) for g in range(3)],
        axis=1)
    whh_p = gates(Whh)
    bhh_p = jnp.concatenate(
        [bhh[:, g * C:(g + 1) * C].reshape(1, K * C) for g in range(3)],
        axis=1)
    return (wc_all, bc2, gmask, wio, bio, wih_p, bih_p, whh_p, bhh_p)


def _dense(A, gathered, packed):
    B = A.shape[0]
    R = _NB * L
    grid = B // _NB
    f32 = jnp.float32

    def wspec(shape):
        nd = len(shape)
        return pl.BlockSpec(shape, lambda i, _n=nd: (0,) * _n)

    in_specs = [
        pl.BlockSpec((_NB, L, 2 * L), lambda i: (i, 0, 0)),
        pl.BlockSpec((R, D), lambda i: (i, 0)),
    ] + [wspec(p.shape) for p in packed]

    out_specs = [
        pl.BlockSpec((_NB, L, D), lambda i: (i, 0, 0)),
        pl.BlockSpec((K, _NB, L, C), lambda i: (0, i, 0, 0)),
    ]

    out, cor = pl.pallas_call(
        _tc_body,
        grid=(grid,),
        in_specs=in_specs,
        out_specs=out_specs,
        out_shape=[
            jax.ShapeDtypeStruct((B, L, D), f32),
            jax.ShapeDtypeStruct((K, B, L, C), f32),
        ],
        scratch_shapes=[
            pltpu.VMEM((R, D), f32),
            pltpu.VMEM((R, 2 * D), f32),
            pltpu.VMEM((R, 2 * D), f32),
        ],
    )(A, gathered, *packed)
    return out, cor


def kernel(inputs, A, emb, Wc, bc, Win, bin_, Wout, bout, Wih, bih, Whh, bhh):
    B, Ls = inputs.shape
    rows = B * Ls
    rpw = rows // _NW
    n_chunk = rpw // _CHUNK
    ids3 = inputs.astype(jnp.int32).reshape(_NW, n_chunk, _CHUNK)
    gathered = _sc_gather(emb, ids3)
    packed = _pack_weights(Wc, bc, Win, bin_, Wout, bout, Wih, bih, Whh, bhh)
    return _dense(A, gathered, packed)


# speedup vs baseline: 2.5762x; 1.0219x over previous
import functools

import jax
import jax.numpy as jnp
from jax import lax
from jax.experimental import pallas as pl
from jax.experimental.pallas import tpu as pltpu
from jax.experimental.pallas import tpu_sc as plsc

D = 64
K = 4
C = 16
L = 50
ITER = 2

_NC = 2
_NS = 16
_NW = _NC * _NS
_CHUNK = 80


def _sc_gather(emb, ids3):
    n_chunk = ids3.shape[1]
    rpw = n_chunk * _CHUNK
    total = _NW * rpw
    mesh = plsc.VectorSubcoreMesh(core_axis_name="c", subcore_axis_name="s")

    @functools.partial(
        pl.kernel,
        mesh=mesh,
        out_type=jax.ShapeDtypeStruct((total, D), jnp.float32),
        scratch_types=[
            pltpu.VMEM((n_chunk, _CHUNK), jnp.int32),
            pltpu.VMEM((rpw, D), jnp.float32),
            pltpu.SemaphoreType.DMA,
        ],
        compiler_params=pltpu.CompilerParams(use_tc_tiling_on_sc=False),
    )
    def gather_kernel(emb_hbm, idx_hbm, out_hbm, idx_v, rows_v, sem):
        wid = lax.axis_index("s") * _NC + lax.axis_index("c")
        pltpu.sync_copy(idx_hbm.at[wid], idx_v)
        copies = []
        for c in range(n_chunk):
            copies.append(
                pltpu.async_copy(
                    emb_hbm.at[idx_v.at[c]],
                    rows_v.at[pl.ds(c * _CHUNK, _CHUNK)],
                    sem,
                )
            )
        for cp in copies:
            cp.wait()
        pltpu.sync_copy(rows_v, out_hbm.at[pl.ds(wid * rpw, rpw)])

    return gather_kernel(emb, ids3)


_NB = 16


def _tc_body(a_ref, hid_ref,
             wc_ref, bc_ref, g_ref,
             wio_ref, bio_ref, wih_ref, bih_ref, whh_ref, bhh_ref,
             out_ref, cor_ref,
             h_sc, p_sc, x_sc):
    f32 = jnp.float32
    bf16 = jnp.bfloat16
    R = _NB * L

    def mm(x, w_ref):
        return jnp.dot(x, w_ref[...], preferred_element_type=f32)

    hid = hid_ref[...].astype(bf16)
    hk = jnp.tanh(mm(hid, wc_ref) + bc_ref[...])
    hk2 = (hk * hk).astype(bf16)
    ssq = jnp.dot(hk2, g_ref[...], preferred_element_type=f32)
    hk = hk * lax.rsqrt(ssq + 1e-12)
    for k in range(K):
        cor_ref[k] = jnp.reshape(hk[:, k * C:(k + 1) * C], (_NB, L, C))
    h_sc[...] = hk

    a_bf = a_ref[...].astype(bf16)
    col = lax.broadcasted_iota(jnp.int32, (L, 2 * D), 1)
    mlo = (col < D).astype(bf16)
    mhi = (col >= D).astype(bf16)
    for _ in range(ITER):
        h = h_sc[...]
        h_bf = h.astype(bf16)
        p_sc[...] = (mm(h_bf, wio_ref) + bio_ref[...]).astype(bf16)
        for s in range(_NB):
            p_s = p_sc[pl.ds(s * L, L), :]
            pp = jnp.concatenate([p_s * mlo, p_s * mhi], axis=0)
            x_sc[pl.ds(s * L, L), :] = jnp.dot(
                a_bf[s], pp, preferred_element_type=f32).astype(bf16)
        gi = mm(x_sc[...], wih_ref) + bih_ref[...]
        gh = mm(h_bf, whh_ref) + bhh_ref[...]
        r = jax.nn.sigmoid(gi[:, :D] + gh[:, :D])
        z = jax.nn.sigmoid(gi[:, D:2 * D] + gh[:, D:2 * D])
        n = jnp.tanh(gi[:, 2 * D:] + r * gh[:, 2 * D:])
        h_sc[...] = (1.0 - z) * n + z * h
    out_ref[...] = jnp.reshape(h_sc[...], (_NB, L, D))


def _block_diag(w):
    eye = jnp.eye(K, dtype=w.dtype)
    t = w[:, :, None, :] * eye[:, None, :, None]
    return t.reshape(K * w.shape[1], K * w.shape[2])


def _pack_weights(Wc, bc, Win, bin_, Wout, bout, Wih, bih, Whh, bhh):
    wc_all = Wc.transpose(1, 0, 2).reshape(D, K * C)
    bc2 = bc.reshape(1, K * C)
    gmask = jnp.kron(jnp.eye(K, dtype=jnp.float32),
                     jnp.ones((C, C), jnp.float32))
    wio = jnp.concatenate([_block_diag(Win), _block_diag(Wout)], axis=1)
    bio = jnp.concatenate([bin_.reshape(1, K * C), bout.reshape(1, K * C)],
                          axis=1)

    def gates(w):
        return jnp.concatenate(
            [_block_diag(w[:, :, g * C:(g + 1) * C]) for g in range(3)],
            axis=1)

    wih_p = jnp.concatenate([gates(Wih[:, :C, :]), gates(Wih[:, C:, :])],
                            axis=0)
    bih_p = jnp.concatenate(
        [bih[:, g * C:(g + 1) * C].reshape(1, K * C) for g in range(3)],
        axis=1)
    whh_p = gates(Whh)
    bhh_p = jnp.concatenate(
        [bhh[:, g * C:(g + 1) * C].reshape(1, K * C) for g in range(3)],
        axis=1)
    bf16 = jnp.bfloat16
    return (wc_all.astype(bf16), bc2, gmask.astype(bf16), wio.astype(bf16),
            bio, wih_p.astype(bf16), bih_p, whh_p.astype(bf16), bhh_p)


def _dense(A, gathered, packed):
    B = A.shape[0]
    R = _NB * L
    grid = B // _NB
    f32 = jnp.float32

    def wspec(shape):
        nd = len(shape)
        return pl.BlockSpec(shape, lambda i, _n=nd: (0,) * _n)

    in_specs = [
        pl.BlockSpec((_NB, L, 2 * L), lambda i: (i, 0, 0)),
        pl.BlockSpec((R, D), lambda i: (i, 0)),
    ] + [wspec(p.shape) for p in packed]

    out_specs = [
        pl.BlockSpec((_NB, L, D), lambda i: (i, 0, 0)),
        pl.BlockSpec((K, _NB, L, C), lambda i: (0, i, 0, 0)),
    ]

    out, cor = pl.pallas_call(
        _tc_body,
        grid=(grid,),
        in_specs=in_specs,
        out_specs=out_specs,
        out_shape=[
            jax.ShapeDtypeStruct((B, L, D), f32),
            jax.ShapeDtypeStruct((K, B, L, C), f32),
        ],
        scratch_shapes=[
            pltpu.VMEM((R, D), f32),
            pltpu.VMEM((R, 2 * D), jnp.bfloat16),
            pltpu.VMEM((R, 2 * D), jnp.bfloat16),
        ],
    )(A, gathered, *packed)
    return out, cor


def kernel(inputs, A, emb, Wc, bc, Win, bin_, Wout, bout, Wih, bih, Whh, bhh):
    B, Ls = inputs.shape
    rows = B * Ls
    rpw = rows // _NW
    n_chunk = rpw // _CHUNK
    ids3 = inputs.astype(jnp.int32).reshape(_NW, n_chunk, _CHUNK)
    gathered = _sc_gather(emb, ids3)
    packed = _pack_weights(Wc, bc, Win, bin_, Wout, bout, Wih, bih, Whh, bhh)
    return _dense(A, gathered, packed)
